# baseline (speedup 1.0000x reference)
import jax
import jax.numpy as jnp
from jax.experimental import pallas as pl

N_NODES = 100000
NUM_TYPES = 128
ROW_BLOCK = 2000


def _onehot_body(idx_ref, out1_ref, out2_ref):
    idx = idx_ref[...]
    classes = jax.lax.broadcasted_iota(jnp.int32, (1, NUM_TYPES), 1)
    oh = jnp.where(idx == classes, jnp.float32(1.0), jnp.float32(0.0))
    out1_ref[...] = oh
    out2_ref[...] = oh


def kernel(elem_map, pos):
    del pos
    grid = (N_NODES // ROW_BLOCK,)
    out_shape = jax.ShapeDtypeStruct((N_NODES, NUM_TYPES), jnp.float32)
    oh1, oh2 = pl.pallas_call(
        _onehot_body,
        grid=grid,
        in_specs=[pl.BlockSpec((ROW_BLOCK, 1), lambda i: (i, 0))],
        out_specs=[
            pl.BlockSpec((ROW_BLOCK, NUM_TYPES), lambda i: (i, 0)),
            pl.BlockSpec((ROW_BLOCK, NUM_TYPES), lambda i: (i, 0)),
        ],
        out_shape=[out_shape, out_shape],
    )(elem_map)
    return (oh1, oh2)


# speedup vs baseline: 1.2441x; 1.2441x over previous
import jax
import jax.numpy as jnp
from jax.experimental import pallas as pl

N_NODES = 100000
NUM_TYPES = 128
ROW_BLOCK = 10000


def _onehot_body(idx_ref, out1_ref, out2_ref):
    idx = idx_ref[...]
    classes = jax.lax.broadcasted_iota(jnp.int32, (1, NUM_TYPES), 1)
    oh = jnp.where(idx == classes, jnp.float32(1.0), jnp.float32(0.0))
    out1_ref[...] = oh
    out2_ref[...] = oh


def kernel(elem_map, pos):
    del pos
    grid = (N_NODES // ROW_BLOCK,)
    out_shape = jax.ShapeDtypeStruct((N_NODES, NUM_TYPES), jnp.float32)
    oh1, oh2 = pl.pallas_call(
        _onehot_body,
        grid=grid,
        in_specs=[pl.BlockSpec((ROW_BLOCK, 1), lambda i: (i, 0))],
        out_specs=[
            pl.BlockSpec((ROW_BLOCK, NUM_TYPES), lambda i: (i, 0)),
            pl.BlockSpec((ROW_BLOCK, NUM_TYPES), lambda i: (i, 0)),
        ],
        out_shape=[out_shape, out_shape],
    )(elem_map)
    return (oh1, oh2)


# speedup vs baseline: 1.2685x; 1.0197x over previous
import jax
import jax.numpy as jnp
from jax.experimental import pallas as pl

N_NODES = 100000
NUM_TYPES = 128
ROW_BLOCK = 20000


def _onehot_body(idx_ref, out1_ref, out2_ref):
    idx = idx_ref[...]
    classes = jax.lax.broadcasted_iota(jnp.int32, (1, NUM_TYPES), 1)
    oh = jnp.where(idx == classes, jnp.float32(1.0), jnp.float32(0.0))
    out1_ref[...] = oh
    out2_ref[...] = oh


def kernel(elem_map, pos):
    del pos
    grid = (N_NODES // ROW_BLOCK,)
    out_shape = jax.ShapeDtypeStruct((N_NODES, NUM_TYPES), jnp.float32)
    oh1, oh2 = pl.pallas_call(
        _onehot_body,
        grid=grid,
        in_specs=[pl.BlockSpec((ROW_BLOCK, 1), lambda i: (i, 0))],
        out_specs=[
            pl.BlockSpec((ROW_BLOCK, NUM_TYPES), lambda i: (i, 0)),
            pl.BlockSpec((ROW_BLOCK, NUM_TYPES), lambda i: (i, 0)),
        ],
        out_shape=[out_shape, out_shape],
    )(elem_map)
    return (oh1, oh2)


# speedup vs baseline: 1.4141x; 1.1147x over previous
import jax
import jax.numpy as jnp
from jax import lax
from jax.experimental import pallas as pl
from jax.experimental.pallas import tpu as pltpu
from jax.experimental.pallas import tpu_sc as plsc

N_NODES = 100000
NUM_TYPES = 128
L = 16
NW = 32
CHUNK = 512
NFULL = N_NODES // CHUNK
TAIL = N_NODES - NFULL * CHUNK
TAIL_BASE = NFULL * CHUNK
CHUNKS_PER_W = -(-NFULL // NW)


def _scatter_groups(buf, idx_v, n_rows, value):
    vals = jnp.full((L,), value, dtype=jnp.float32)
    row_off = lax.broadcasted_iota(jnp.int32, (L,), 0) * NUM_TYPES
    for g in range(n_rows // L):
        cols = idx_v[pl.ds(g * L, L)]
        flat = row_off + (g * L * NUM_TYPES) + cols
        plsc.store_scatter(buf, [flat], vals)


def _sc_body(elem_hbm, out1_hbm, out2_hbm, idx_v, buf, sem1, sem2):
    wid = lax.axis_index("s") * 2 + lax.axis_index("c")

    zeros = jnp.zeros((L,), jnp.float32)

    def _zero_step(k, _):
        for j in range(8):
            buf[pl.ds(k * 8 * L + j * L, L)] = zeros
        return 0

    lax.fori_loop(0, CHUNK * NUM_TYPES // (8 * L), _zero_step, 0)

    def _do_chunk(base, n_rows):
        pltpu.sync_copy(elem_hbm.at[pl.ds(base, n_rows)], idx_v.at[pl.ds(0, n_rows)])
        _scatter_groups(buf, idx_v, n_rows, 1.0)
        c1 = pltpu.async_copy(buf.at[pl.ds(0, n_rows * NUM_TYPES)],
                              out1_hbm.at[pl.ds(base * NUM_TYPES, n_rows * NUM_TYPES)],
                              sem1)
        c2 = pltpu.async_copy(buf.at[pl.ds(0, n_rows * NUM_TYPES)],
                              out2_hbm.at[pl.ds(base * NUM_TYPES, n_rows * NUM_TYPES)],
                              sem2)
        c1.wait()
        c2.wait()
        _scatter_groups(buf, idx_v, n_rows, 0.0)

    def _chunk_step(i, _):
        c = wid + i * NW

        @pl.when(c < NFULL)
        def _():
            _do_chunk(c * CHUNK, CHUNK)

        return 0

    lax.fori_loop(0, CHUNKS_PER_W, _chunk_step, 0)

    @pl.when(wid == 3)
    def _():
        _do_chunk(TAIL_BASE, TAIL)


def kernel(elem_map, pos):
    del pos
    out_sds = jax.ShapeDtypeStruct((N_NODES * NUM_TYPES,), jnp.float32)
    mesh = plsc.VectorSubcoreMesh(core_axis_name="c", subcore_axis_name="s")
    sc_call = pl.kernel(
        _sc_body,
        out_type=(out_sds, out_sds),
        mesh=mesh,
        compiler_params=pltpu.CompilerParams(needs_layout_passes=False),
        scratch_types=[
            pltpu.VMEM((CHUNK,), jnp.int32),
            pltpu.VMEM((CHUNK * NUM_TYPES,), jnp.float32),
            pltpu.SemaphoreType.DMA,
            pltpu.SemaphoreType.DMA,
        ],
    )
    oh1, oh2 = sc_call(jnp.reshape(elem_map, (N_NODES,)))
    shape2d = (N_NODES, NUM_TYPES)
    return (jnp.reshape(oh1, shape2d), jnp.reshape(oh2, shape2d))
